# baseline (speedup 1.0000x reference)
import jax
import jax.numpy as jnp
from jax.experimental import pallas as pl
from jax.experimental.pallas import tpu as pltpu
from jax.experimental.pallas import tpu_sc as plsc

K = 512
D = 16
EPS = 1e-15
_PREC = jax.lax.Precision.HIGHEST
_CHUNK = 256
_NTILES = 13
_NPAD = _NTILES * _CHUNK
_L = 16


def _dot(a, b):
    return jax.lax.dot_general(a, b, ((((1,), (0,))), ((), ())),
                               preferred_element_type=jnp.float32,
                               precision=_PREC)


def _assign_kernel(x_ref, e_ref, idx_ref, q_ref, vat_ref, kta_scr):
    nh, hw = x_ref.shape[0], x_ref.shape[2]
    n = nh * hw
    vat_ref[...] = jnp.zeros_like(vat_ref)
    for h in range(nh):
        blk = x_ref[h]
        q_ref[pl.ds(h * hw, hw), :] = jnp.transpose(blk[0:D, :])
        kta_scr[0:D, pl.ds(h * hw, hw)] = blk[D:2 * D, :]
        vat_ref[0:D, pl.ds(h * hw, hw)] = blk[2 * D:3 * D, :]
    kta_scr[D:D + 1, :] = jnp.ones_like(kta_scr[D:D + 1, :])
    vat_ref[D:D + 1, 0:n] = jnp.ones_like(vat_ref[D:D + 1, 0:n])

    e = e_ref[...]
    e_sq = jnp.sum(e * e, axis=1, keepdims=True)
    e_aug = jnp.concatenate([e * (-2.0), e_sq], axis=1)
    dist_t = _dot(e_aug, kta_scr[...])

    min_d = jnp.min(dist_t, axis=0, keepdims=True)
    code_iota = jax.lax.broadcasted_iota(jnp.int32, dist_t.shape, 0)
    idx = jnp.min(jnp.where(dist_t == min_d, code_iota, K),
                  axis=0, keepdims=True)
    idx_ref[:, 0:n] = idx
    idx_ref[:, n:] = jnp.full_like(idx_ref[:, n:], K)


def _sc_scatter_body(idx_hbm, vat_hbm, zero_hbm, out_hbm, idx_v, cols_v, acc):
    c = jax.lax.axis_index("c")
    s = jax.lax.axis_index("s")
    wid = s * 2 + c

    @pl.when(wid < _NTILES)
    def _():
        base = wid * _CHUNK
        pltpu.sync_copy(idx_hbm.at[pl.ds(base, _CHUNK)], idx_v)
        pltpu.sync_copy(vat_hbm.at[:, pl.ds(base, _CHUNK)], cols_v)
        pltpu.sync_copy(zero_hbm, acc)
        for g in range(_CHUNK // _L):
            rows = idx_v[pl.ds(g * _L, _L)]
            flat = rows * 32
            for col in range(32):
                vals = cols_v[col, pl.ds(g * _L, _L)]
                plsc.addupdate_scatter(
                    acc, [flat + jnp.full((_L,), col, jnp.int32)], vals)
        pltpu.sync_copy(acc, out_hbm.at[wid])


def _sc_scatter(idx, vat, zero):
    return pl.kernel(
        _sc_scatter_body,
        out_type=jax.ShapeDtypeStruct((_NTILES, (K + 1) * 32), jnp.float32),
        mesh=plsc.VectorSubcoreMesh(core_axis_name="c", subcore_axis_name="s",
                                    num_cores=2, num_subcores=16),
        scratch_types=[
            pltpu.VMEM((_CHUNK,), jnp.int32),
            pltpu.VMEM((32, _CHUNK), jnp.float32),
            pltpu.VMEM(((K + 1) * 32,), jnp.float32),
        ],
        compiler_params=pltpu.CompilerParams(needs_layout_passes=False),
    )(idx, vat, zero)


def _combine_kernel(q_ref, e_ref, otv_ref, o_ref):
    otv_aug = otv_ref[0, 0:K, 0:D + 1]
    for t in range(1, _NTILES):
        otv_aug = otv_aug + otv_ref[t, 0:K, 0:D + 1]
    p = jnp.exp(_dot(q_ref[...], jnp.transpose(e_ref[...])))
    num_aug = _dot(p, otv_aug)
    o_ref[...] = num_aug[:, :D] / (num_aug[:, D:D + 1] + EPS)


def kernel(qkv, embedding_weight):
    B, C, H, W = qkv.shape
    nh = B * C // (3 * D)
    n = nh * H * W
    x = jnp.reshape(qkv.astype(jnp.float32), (nh, 3 * D, H * W))
    e = embedding_weight.astype(jnp.float32)

    idx, q, vat = pl.pallas_call(
        _assign_kernel,
        out_shape=[
            jax.ShapeDtypeStruct((1, _NPAD), jnp.int32),
            jax.ShapeDtypeStruct((n, D), jnp.float32),
            jax.ShapeDtypeStruct((32, _NPAD), jnp.float32),
        ],
        scratch_shapes=[pltpu.VMEM((D + 1, n), jnp.float32)],
    )(x, e)

    otv13 = _sc_scatter(jnp.reshape(idx, (_NPAD,)), vat,
                        jnp.zeros(((K + 1) * 32,), jnp.float32))
    otv13 = jnp.reshape(otv13, (_NTILES, K + 1, 32))

    out = pl.pallas_call(
        _combine_kernel,
        out_shape=jax.ShapeDtypeStruct((n, D), jnp.float32),
    )(q, e, otv13)
    return jnp.reshape(out, (B, -1, H, W))


# speedup vs baseline: 2.8076x; 2.8076x over previous
import jax
import jax.numpy as jnp
from jax.experimental import pallas as pl
from jax.experimental.pallas import tpu as pltpu

K = 512
D = 16
EPS = 1e-15
_PREC = jax.lax.Precision.HIGHEST


def _dot(a, b, prec=_PREC):
    return jax.lax.dot_general(a, b, ((((1,), (0,))), ((), ())),
                               preferred_element_type=jnp.float32,
                               precision=prec)


def _vq_attn_kernel(x_ref, e_ref, o_ref, q_scr, kta_scr, va_scr):
    nh = x_ref.shape[0]
    hw = x_ref.shape[2]
    for h in range(nh):
        blk = x_ref[h]
        q_scr[pl.ds(h * hw, hw), :] = jnp.transpose(blk[0:D, :])
        kta_scr[0:D, pl.ds(h * hw, hw)] = blk[D:2 * D, :]
        va_scr[pl.ds(h * hw, hw), 0:D] = jnp.transpose(blk[2 * D:3 * D, :])
    kta_scr[D:D + 1, :] = jnp.ones_like(kta_scr[D:D + 1, :])
    va_scr[:, D:D + 1] = jnp.ones_like(va_scr[:, D:D + 1])

    e = e_ref[...]
    e_sq = jnp.sum(e * e, axis=1, keepdims=True)
    e_aug = jnp.concatenate([e * (-2.0), e_sq], axis=1)

    dist_t = _dot(e_aug, kta_scr[...])

    min_d = jnp.min(dist_t, axis=0, keepdims=True)
    code_iota = jax.lax.broadcasted_iota(jnp.int32, dist_t.shape, 0)
    idx = jnp.min(jnp.where(dist_t == min_d, code_iota, K),
                  axis=0, keepdims=True)
    one_hot_t = (code_iota == idx).astype(jnp.float32)

    otv_aug = _dot(one_hot_t, va_scr[...], jax.lax.Precision.DEFAULT)

    p = jnp.exp(_dot(q_scr[...], jnp.transpose(e)))
    num_aug = _dot(p, otv_aug, jax.lax.Precision.DEFAULT)
    o_ref[...] = num_aug[:, :D] / (num_aug[:, D:D + 1] + EPS)


def kernel(qkv, embedding_weight):
    B, C, H, W = qkv.shape
    nh = B * C // (3 * D)
    n = nh * H * W
    x = jnp.reshape(qkv.astype(jnp.float32), (nh, 3 * D, H * W))

    out = pl.pallas_call(
        _vq_attn_kernel,
        out_shape=jax.ShapeDtypeStruct((n, D), jnp.float32),
        scratch_shapes=[
            pltpu.VMEM((n, D), jnp.float32),
            pltpu.VMEM((D + 1, n), jnp.float32),
            pltpu.VMEM((n, D + 1), jnp.float32),
        ],
    )(x, embedding_weight.astype(jnp.float32))
    return jnp.reshape(out, (B, -1, H, W))


# speedup vs baseline: 3.4007x; 1.2113x over previous
import jax
import jax.numpy as jnp
from jax.experimental import pallas as pl
from jax.experimental.pallas import tpu as pltpu

K = 512
D = 16
EPS = 1e-15
_PREC = jax.lax.Precision.HIGHEST


def _dot(a, b, prec=_PREC):
    return jax.lax.dot_general(a, b, ((((1,), (0,))), ((), ())),
                               preferred_element_type=jnp.float32,
                               precision=prec)


def _vq_attn_kernel(x_ref, e_ref, o_ref, q_scr, kta_scr, va_scr):
    nh = x_ref.shape[0]
    hw = x_ref.shape[2]
    for h in range(nh):
        blk = x_ref[h]
        q_scr[pl.ds(h * hw, hw), :] = jnp.transpose(blk[0:D, :])
        kta_scr[0:D, pl.ds(h * hw, hw)] = blk[D:2 * D, :]
        va_scr[pl.ds(h * hw, hw), 0:D] = jnp.transpose(blk[2 * D:3 * D, :])
    kta_scr[D:D + 1, :] = jnp.ones_like(kta_scr[D:D + 1, :])
    va_scr[:, D:D + 1] = jnp.ones_like(va_scr[:, D:D + 1])

    e = e_ref[...]
    e_sq = jnp.sum(e * e, axis=1, keepdims=True)
    e_aug = jnp.concatenate([e * (-2.0), e_sq], axis=1)

    dist_t = _dot(e_aug, kta_scr[...])

    min_d = jnp.min(dist_t, axis=0, keepdims=True)
    code_iota = jax.lax.broadcasted_iota(jnp.int32, dist_t.shape, 0)
    idx = jnp.min(jnp.where(dist_t == min_d, code_iota, K),
                  axis=0, keepdims=True)
    one_hot_t = (code_iota == idx).astype(jnp.float32)

    otv_aug = _dot(one_hot_t, va_scr[...], jax.lax.Precision.DEFAULT)

    p = jnp.exp(_dot(q_scr[...], jnp.transpose(e), jax.lax.Precision.DEFAULT))
    num_aug = _dot(p, otv_aug, jax.lax.Precision.DEFAULT)
    o_ref[...] = num_aug[:, :D] / (num_aug[:, D:D + 1] + EPS)


def kernel(qkv, embedding_weight):
    B, C, H, W = qkv.shape
    nh = B * C // (3 * D)
    n = nh * H * W
    x = jnp.reshape(qkv.astype(jnp.float32), (nh, 3 * D, H * W))

    out = pl.pallas_call(
        _vq_attn_kernel,
        out_shape=jax.ShapeDtypeStruct((n, D), jnp.float32),
        scratch_shapes=[
            pltpu.VMEM((n, D), jnp.float32),
            pltpu.VMEM((D + 1, n), jnp.float32),
            pltpu.VMEM((n, D + 1), jnp.float32),
        ],
    )(x, embedding_weight.astype(jnp.float32))
    return jnp.reshape(out, (B, -1, H, W))
